# Initial kernel scaffold; baseline (speedup 1.0000x reference)
import jax
import jax.numpy as jnp
from jax.experimental import pallas as pl


def kernel(car_x, infra_x, car_angles, infra_angles, car_pos, infra_pos, car_edge_index, infra_edge_index, match_idx, Wn_car, We_car, a_car, Wn_infra, We_infra, a_infra, W_gate, b_gate):
    raise NotImplementedError("write your pallas kernel here")


# speedup vs baseline: 8.1662x; 8.1662x over previous
import functools

import jax
import jax.numpy as jnp
from jax.experimental import pallas as pl

N_NODES = 50000
E_EDGES = 800000
HEADS = 8
EMBED = 16
D = 128
RADIUS = 50.0

_NB = 5000
_EB = 2000


def _prep_body(x_ref, ang_ref, p_ref, xr_ref, a_ref):
    x = x_ref[...]
    ang = ang_ref[...]
    c = jnp.cos(ang)
    s = jnp.sin(ang)
    x0 = x[:, 0:1]
    x1 = x[:, 1:2]
    xr0 = x0 * c + x1 * s
    xr1 = -x0 * s + x1 * c
    xr = jnp.concatenate([xr0, xr1], axis=1)
    xr_ref[...] = xr
    a_ref[...] = jnp.dot(xr, p_ref[...], preferred_element_type=jnp.float32)


def _prep(x, ang, p):
    nb = N_NODES // _NB
    return pl.pallas_call(
        _prep_body,
        grid=(nb,),
        in_specs=[
            pl.BlockSpec((_NB, 2), lambda i: (i, 0)),
            pl.BlockSpec((_NB, 1), lambda i: (i, 0)),
            pl.BlockSpec((2, 16), lambda i: (0, 0)),
        ],
        out_specs=[
            pl.BlockSpec((_NB, 2), lambda i: (i, 0)),
            pl.BlockSpec((_NB, 16), lambda i: (i, 0)),
        ],
        out_shape=[
            jax.ShapeDtypeStruct((N_NODES, 2), jnp.float32),
            jax.ShapeDtypeStruct((N_NODES, 16), jnp.float32),
        ],
    )(x, ang, p)


def _logits_body(a0s_ref, a1d_ref, ps_ref, pd_ref, wa_ref, lg_ref, ea_ref):
    ea = ps_ref[...] - pd_ref[...]
    ea_ref[...] = ea
    dist = jnp.sqrt(ea[:, 0:1] * ea[:, 0:1] + ea[:, 1:2] * ea[:, 1:2])
    keep = dist < RADIUS
    z = a0s_ref[...] + a1d_ref[...] + jnp.dot(
        ea, wa_ref[...], preferred_element_type=jnp.float32)
    z = jnp.where(z >= 0.0, z, 0.2 * z)
    lg_ref[...] = jnp.where(keep, z, -1e9)


def _edge_logits(a0s, a1d, ps, pd, wa):
    nb = E_EDGES // _EB
    return pl.pallas_call(
        _logits_body,
        grid=(nb,),
        in_specs=[
            pl.BlockSpec((_EB, 8), lambda i: (i, 0)),
            pl.BlockSpec((_EB, 8), lambda i: (i, 0)),
            pl.BlockSpec((_EB, 2), lambda i: (i, 0)),
            pl.BlockSpec((_EB, 2), lambda i: (i, 0)),
            pl.BlockSpec((2, 8), lambda i: (0, 0)),
        ],
        out_specs=[
            pl.BlockSpec((_EB, 8), lambda i: (i, 0)),
            pl.BlockSpec((_EB, 2), lambda i: (i, 0)),
        ],
        out_shape=[
            jax.ShapeDtypeStruct((E_EDGES, 8), jnp.float32),
            jax.ShapeDtypeStruct((E_EDGES, 2), jnp.float32),
        ],
    )(a0s, a1d, ps, pd, wa)


def _ex_body(lg_ref, md_ref, ea_ref, ex_ref):
    ea = ea_ref[...]
    dist = jnp.sqrt(ea[:, 0:1] * ea[:, 0:1] + ea[:, 1:2] * ea[:, 1:2])
    keep = (dist < RADIUS).astype(jnp.float32)
    ex_ref[...] = jnp.exp(lg_ref[...] - md_ref[...]) * keep


def _edge_ex(lg, md, ea):
    nb = E_EDGES // _EB
    return pl.pallas_call(
        _ex_body,
        grid=(nb,),
        in_specs=[
            pl.BlockSpec((_EB, 8), lambda i: (i, 0)),
            pl.BlockSpec((_EB, 8), lambda i: (i, 0)),
            pl.BlockSpec((_EB, 2), lambda i: (i, 0)),
        ],
        out_specs=pl.BlockSpec((_EB, 8), lambda i: (i, 0)),
        out_shape=jax.ShapeDtypeStruct((E_EDGES, 8), jnp.float32),
    )(lg, md, ea)


def _u_body(ex_ref, dd_ref, xrs_ref, ea_ref, u_ref):
    alpha = ex_ref[...] / dd_ref[...]
    feat = jnp.concatenate([xrs_ref[...], ea_ref[...]], axis=1)
    parts = [alpha[:, h:h + 1] * feat for h in range(HEADS)]
    u_ref[...] = jnp.concatenate(parts, axis=1)


def _edge_u(ex, dd, xrs, ea):
    nb = E_EDGES // _EB
    return pl.pallas_call(
        _u_body,
        grid=(nb,),
        in_specs=[
            pl.BlockSpec((_EB, 8), lambda i: (i, 0)),
            pl.BlockSpec((_EB, 8), lambda i: (i, 0)),
            pl.BlockSpec((_EB, 2), lambda i: (i, 0)),
            pl.BlockSpec((_EB, 2), lambda i: (i, 0)),
        ],
        out_specs=pl.BlockSpec((_EB, 32), lambda i: (i, 0)),
        out_shape=jax.ShapeDtypeStruct((E_EDGES, 32), jnp.float32),
    )(ex, dd, xrs, ea)


def _fin_body(ts_ref, wn_ref, we_ref, out_ref):
    r = jax.lax.broadcasted_iota(jnp.int32, (32, 128), 0)
    col = jax.lax.broadcasted_iota(jnp.int32, (32, 128), 1)
    h_r = r // 4
    k = r % 4
    h_c = col // 16
    wn = wn_ref[...]
    we = we_ref[...]
    base = jnp.where(
        k == 0, wn[0:1, :],
        jnp.where(k == 1, wn[1:2, :],
                  jnp.where(k == 2, we[0:1, :], we[1:2, :])))
    m = jnp.where(h_r == h_c, base, 0.0)
    out_ref[...] = jnp.dot(ts_ref[...], m, preferred_element_type=jnp.float32)


def _finalize(ts, wn, we):
    nb = N_NODES // _NB
    return pl.pallas_call(
        _fin_body,
        grid=(nb,),
        in_specs=[
            pl.BlockSpec((_NB, 32), lambda i: (i, 0)),
            pl.BlockSpec((2, 128), lambda i: (0, 0)),
            pl.BlockSpec((2, 128), lambda i: (0, 0)),
        ],
        out_specs=pl.BlockSpec((_NB, 128), lambda i: (i, 0)),
        out_shape=jax.ShapeDtypeStruct((N_NODES, 128), jnp.float32),
    )(ts, wn, we)


def _fuse_body(ch_ref, ig_ref, wg1_ref, wg2_ref, b_ref, out_ref):
    ch = ch_ref[...]
    ig = ig_ref[...]
    z = (jnp.dot(ch, wg1_ref[...], preferred_element_type=jnp.float32)
         + jnp.dot(ig, wg2_ref[...], preferred_element_type=jnp.float32)
         + b_ref[...])
    gate = 1.0 / (1.0 + jnp.exp(-z))
    out_ref[...] = gate * ch + (1.0 - gate) * ig


def _fuse(ch, ig, wg1, wg2, b):
    nb = N_NODES // _NB
    return pl.pallas_call(
        _fuse_body,
        grid=(nb,),
        in_specs=[
            pl.BlockSpec((_NB, 128), lambda i: (i, 0)),
            pl.BlockSpec((_NB, 128), lambda i: (i, 0)),
            pl.BlockSpec((128, 128), lambda i: (0, 0)),
            pl.BlockSpec((128, 128), lambda i: (0, 0)),
            pl.BlockSpec((1, 128), lambda i: (0, 0)),
        ],
        out_specs=pl.BlockSpec((_NB, 128), lambda i: (i, 0)),
        out_shape=jax.ShapeDtypeStruct((N_NODES, 128), jnp.float32),
    )(ch, ig, wg1, wg2, b)


def _gat_half(x, ang, pos, edge_index, wn, we, a):
    wn3 = wn.reshape(2, HEADS, EMBED)
    we3 = we.reshape(2, HEADS, EMBED)
    p0 = jnp.einsum('khd,hd->kh', wn3, a[0])
    p1 = jnp.einsum('khd,hd->kh', wn3, a[1])
    wa = jnp.einsum('khd,hd->kh', we3, a[2])
    p01 = jnp.concatenate([p0, p1], axis=1)

    xr, a01 = _prep(x, ang.reshape(-1, 1), p01)
    src = edge_index[0]
    dst = edge_index[1]

    a0s = jnp.take(a01[:, :8], src, axis=0)
    a1d = jnp.take(a01[:, 8:], dst, axis=0)
    ps = jnp.take(pos, src, axis=0)
    pd = jnp.take(pos, dst, axis=0)

    lg, ea = _edge_logits(a0s, a1d, ps, pd, wa)

    m = jax.ops.segment_max(lg, dst, num_segments=N_NODES)
    m = jnp.where(jnp.isfinite(m), m, 0.0)
    md = jnp.take(m, dst, axis=0)
    ex = _edge_ex(lg, md, ea)

    denom = jax.ops.segment_sum(ex, dst, num_segments=N_NODES) + 1e-16
    dd = jnp.take(denom, dst, axis=0)
    xrs = jnp.take(xr, src, axis=0)
    u = _edge_u(ex, dd, xrs, ea)

    ts = jax.ops.segment_sum(u, dst, num_segments=N_NODES)
    return _finalize(ts, wn, we)


@jax.jit
def kernel(car_x, infra_x, car_angles, infra_angles, car_pos, infra_pos,
           car_edge_index, infra_edge_index, match_idx,
           Wn_car, We_car, a_car, Wn_infra, We_infra, a_infra,
           W_gate, b_gate):
    car_h = _gat_half(car_x, car_angles, car_pos, car_edge_index,
                      Wn_car, We_car, a_car)
    infra_h = _gat_half(infra_x, infra_angles, infra_pos, infra_edge_index,
                        Wn_infra, We_infra, a_infra)
    infra_g = jnp.take(infra_h, match_idx, axis=0)
    return _fuse(car_h, infra_g, W_gate[:128], W_gate[128:],
                 b_gate.reshape(1, 128))
